# baseline (speedup 1.0000x reference)
import functools

import jax
import jax.numpy as jnp
from jax import lax
from jax.experimental import pallas as pl
from jax.experimental.pallas import tpu as pltpu
from jax.experimental.pallas import tpu_sc as plsc

_B = 4096
_T = 50
_D = 64
_NW = 32
_BT = _B // 128

_mesh = plsc.VectorSubcoreMesh(core_axis_name="c", subcore_axis_name="s")


@functools.partial(
    pl.kernel,
    mesh=_mesh,
    out_type=jax.ShapeDtypeStruct((_T, _D // 8, _BT, 8, 128), jnp.float32),
    scratch_types=[
        pltpu.VMEM((128,), jnp.int32),
        pltpu.VMEM((128, _D), jnp.float32),
        pltpu.VMEM((_D // 8, 8, 128), jnp.float32),
        pltpu.SemaphoreType.DMA,
        pltpu.SemaphoreType.DMA,
    ],
    compiler_params=pltpu.CompilerParams(
        use_tc_tiling_on_sc=False, needs_layout_passes=False
    ),
)
def _lookup_kernel(cap_hbm, tbl_hbm, out_hbm, idx_v, rows_v, obuf_v,
                   gsem, osem):
    w = lax.axis_index("s") * 2 + lax.axis_index("c")
    riota = lax.iota(jnp.int32, 16)

    def body(t, carry):
        pltpu.sync_copy(cap_hbm.at[t, pl.ds(w * 128, 128)], idx_v)
        gcp = pltpu.make_async_copy(tbl_hbm.at[idx_v], rows_v, gsem)
        gcp.start()
        gcp.wait()
        for g in range(8):
            rk = riota + (16 * g)
            for d in range(_D):
                vals = plsc.load_gather(rows_v, [rk, jnp.full((16,), d, jnp.int32)])
                obuf_v[d // 8, d % 8, pl.ds(16 * g, 16)] = vals
        ocp = pltpu.make_async_copy(obuf_v, out_hbm.at[t, :, w], osem)
        ocp.start()
        ocp.wait()
        return carry

    lax.fori_loop(0, _T, body, 0)


def kernel(captions, lengths, emb_weight):
    cap_t = captions.T
    out5d = _lookup_kernel(cap_t, emb_weight)
    x = out5d.transpose(2, 4, 0, 1, 3).reshape(_B, _T, _D)
    return x, lengths


# speedup vs baseline: 1.0736x; 1.0736x over previous
import functools

import jax
import jax.numpy as jnp
from jax import lax
from jax.experimental import pallas as pl
from jax.experimental.pallas import tpu as pltpu
from jax.experimental.pallas import tpu_sc as plsc

_B = 4096
_T = 50
_D = 64
_NW = 32
_BT = _B // 128

_mesh = plsc.VectorSubcoreMesh(core_axis_name="c", subcore_axis_name="s")


@functools.partial(
    pl.kernel,
    mesh=_mesh,
    out_type=jax.ShapeDtypeStruct((_T, _D // 8, _BT, 8, 128), jnp.float32),
    scratch_types=[
        [pltpu.VMEM((128,), jnp.int32) for _ in range(2)],
        [pltpu.VMEM((128, _D), jnp.float32) for _ in range(2)],
        [pltpu.VMEM((_D // 8, 8, 128), jnp.float32) for _ in range(2)],
        pltpu.SemaphoreType.DMA((2,)),
        pltpu.SemaphoreType.DMA((2,)),
    ],
    compiler_params=pltpu.CompilerParams(
        use_tc_tiling_on_sc=False,
        needs_layout_passes=False,
        disable_bounds_checks=True,
    ),
)
def _lookup_kernel(cap_hbm, tbl_hbm, out_hbm, idx_v, rows_v, obuf_v,
                   gsem, osem):
    w = lax.axis_index("s") * 2 + lax.axis_index("c")
    riota = lax.iota(jnp.int32, 16)
    rks = [riota + 16 * g for g in range(8)]

    def stage(t, b):
        pltpu.sync_copy(cap_hbm.at[t, pl.ds(w * 128, 128)], idx_v[b])
        pltpu.make_async_copy(
            tbl_hbm.at[idx_v[b]], rows_v[b], gsem.at[b]
        ).start()

    def gwait(b):
        pltpu.make_async_copy(
            tbl_hbm.at[idx_v[b]], rows_v[b], gsem.at[b]
        ).wait()

    def owait(b):
        pltpu.make_async_copy(
            obuf_v[b], out_hbm.at[0, :, w], osem.at[b]
        ).wait()

    def work(t, b):
        def dbody(d, carry):
            dt = lax.shift_right_logical(d, 3)
            ds_ = d & 7
            dvec = jnp.full((16,), 0, jnp.int32) + d
            for g in range(8):
                vals = plsc.load_gather(rows_v[b], [rks[g], dvec])
                obuf_v[b][dt, ds_, pl.ds(16 * g, 16)] = vals
            return carry

        lax.fori_loop(0, _D, dbody, 0)
        pltpu.make_async_copy(
            obuf_v[b], out_hbm.at[t, :, w], osem.at[b]
        ).start()

    stage(0, 0)

    def body(g, carry):
        t = 2 * g
        stage(t + 1, 1)
        gwait(0)

        @pl.when(g >= 1)
        def _():
            owait(0)

        work(t, 0)

        @pl.when(g < _T // 2 - 1)
        def _():
            stage(t + 2, 0)

        gwait(1)

        @pl.when(g >= 1)
        def _():
            owait(1)

        work(t + 1, 1)
        return carry

    lax.fori_loop(0, _T // 2, body, 0)
    owait(0)
    owait(1)


def kernel(captions, lengths, emb_weight):
    cap_t = captions.T
    out5d = _lookup_kernel(cap_t, emb_weight)
    x = out5d.transpose(2, 4, 0, 1, 3).reshape(_B, _T, _D)
    return x, lengths


# speedup vs baseline: 1.1608x; 1.0812x over previous
import functools

import jax
import jax.numpy as jnp
from jax import lax
from jax.experimental import pallas as pl
from jax.experimental.pallas import tpu as pltpu
from jax.experimental.pallas import tpu_sc as plsc

_B = 4096
_T = 50
_D = 64
_NW = 32
_BT = _B // 128

_mesh = plsc.VectorSubcoreMesh(core_axis_name="c", subcore_axis_name="s")


@functools.partial(
    pl.kernel,
    mesh=_mesh,
    out_type=jax.ShapeDtypeStruct((_T, _D // 8, _BT, 8, 128), jnp.float32),
    scratch_types=[
        [pltpu.VMEM((128,), jnp.int32) for _ in range(2)],
        [pltpu.VMEM((128, _D), jnp.float32) for _ in range(2)],
        [pltpu.VMEM((_D // 8, 8, 128), jnp.float32) for _ in range(2)],
        pltpu.SemaphoreType.DMA((2,)),
        pltpu.SemaphoreType.DMA((2,)),
    ],
    compiler_params=pltpu.CompilerParams(
        use_tc_tiling_on_sc=False,
        needs_layout_passes=False,
        disable_bounds_checks=True,
    ),
)
def _lookup_kernel(cap_hbm, tbl_hbm, out_hbm, idx_v, rows_v, obuf_v,
                   gsem, osem):
    w = lax.axis_index("s") * 2 + lax.axis_index("c")
    riota = lax.iota(jnp.int32, 16)
    rks = [riota + 16 * g for g in range(8)]

    def stage(t, b):
        pltpu.sync_copy(cap_hbm.at[t, pl.ds(w * 128, 128)], idx_v[b])
        pltpu.make_async_copy(
            tbl_hbm.at[idx_v[b]], rows_v[b], gsem.at[b]
        ).start()

    def gwait(b):
        pltpu.make_async_copy(
            tbl_hbm.at[idx_v[b]], rows_v[b], gsem.at[b]
        ).wait()

    def owait(b):
        pltpu.make_async_copy(
            obuf_v[b], out_hbm.at[0, :, w], osem.at[b]
        ).wait()

    def work(t, b):
        def dbody(i, carry):
            vals = []
            for u in range(2):
                d = 2 * i + u
                dvec = jnp.full((16,), 0, jnp.int32) + d
                vals.append(
                    [plsc.load_gather(rows_v[b], [rks[g], dvec])
                     for g in range(8)]
                )
            for u in range(2):
                d = 2 * i + u
                dt = lax.shift_right_logical(d, 3)
                ds_ = d & 7
                for g in range(8):
                    obuf_v[b][dt, ds_, pl.ds(16 * g, 16)] = vals[u][g]
            return carry

        lax.fori_loop(0, _D // 2, dbody, 0)
        pltpu.make_async_copy(
            obuf_v[b], out_hbm.at[t, :, w], osem.at[b]
        ).start()

    stage(0, 0)

    def body(g, carry):
        t = 2 * g
        stage(t + 1, 1)
        gwait(0)

        @pl.when(g >= 1)
        def _():
            owait(0)

        work(t, 0)

        @pl.when(g < _T // 2 - 1)
        def _():
            stage(t + 2, 0)

        gwait(1)

        @pl.when(g >= 1)
        def _():
            owait(1)

        work(t + 1, 1)
        return carry

    lax.fori_loop(0, _T // 2, body, 0)
    owait(0)
    owait(1)


def kernel(captions, lengths, emb_weight):
    cap_t = captions.T
    out5d = _lookup_kernel(cap_t, emb_weight)
    x = out5d.transpose(2, 4, 0, 1, 3).reshape(_B, _T, _D)
    return x, lengths


# speedup vs baseline: 1.1912x; 1.0262x over previous
import functools

import jax
import jax.numpy as jnp
from jax import lax
from jax.experimental import pallas as pl
from jax.experimental.pallas import tpu as pltpu
from jax.experimental.pallas import tpu_sc as plsc

_B = 4096
_T = 50
_D = 64
_NW = 32
_BT = _B // 128
_NB = 5

_mesh = plsc.VectorSubcoreMesh(core_axis_name="c", subcore_axis_name="s")


@functools.partial(
    pl.kernel,
    mesh=_mesh,
    out_type=jax.ShapeDtypeStruct((_T, _D // 8, _BT, 8, 128), jnp.float32),
    scratch_types=[
        pltpu.VMEM((_T, 128), jnp.int32),
        [pltpu.VMEM((128, _D), jnp.float32) for _ in range(_NB)],
        [pltpu.VMEM((_D // 8, 8, 128), jnp.float32) for _ in range(_NB)],
        pltpu.SemaphoreType.DMA,
        pltpu.SemaphoreType.DMA((_NB,)),
        pltpu.SemaphoreType.DMA((_NB,)),
    ],
    compiler_params=pltpu.CompilerParams(
        use_tc_tiling_on_sc=False,
        needs_layout_passes=False,
        disable_bounds_checks=True,
    ),
)
def _lookup_kernel(cap_hbm, tbl_hbm, out_hbm, idx_v, rows_v, obuf_v,
                   isem, gsem, osem):
    w = lax.axis_index("s") * 2 + lax.axis_index("c")
    riota = lax.iota(jnp.int32, 16)
    rks = [riota + 16 * g for g in range(8)]

    pltpu.make_async_copy(
        cap_hbm.at[:, pl.ds(w * 128, 128)], idx_v, isem
    ).start()
    pltpu.make_async_copy(
        cap_hbm.at[:, pl.ds(w * 128, 128)], idx_v, isem
    ).wait()

    def gcopy(t, u):
        return pltpu.make_async_copy(
            tbl_hbm.at[idx_v.at[t]], rows_v[u], gsem.at[u]
        )

    def ocopy(t, u):
        return pltpu.make_async_copy(
            obuf_v[u], out_hbm.at[t, :, w], osem.at[u]
        )

    def transpose(u):
        def dbody(i, carry):
            vals = []
            for k in range(2):
                d = 2 * i + k
                dvec = jnp.full((16,), 0, jnp.int32) + d
                vals.append(
                    [plsc.load_gather(rows_v[u], [rks[g], dvec])
                     for g in range(8)]
                )
            for k in range(2):
                d = 2 * i + k
                dt = lax.shift_right_logical(d, 3)
                ds_ = d & 7
                for g in range(8):
                    obuf_v[u][dt, ds_, pl.ds(16 * g, 16)] = vals[k][g]
            return carry

        lax.fori_loop(0, _D // 2, dbody, 0)

    for t in range(_NB - 1):
        gcopy(t, t).start()

    def body(o, carry):
        for u in range(_NB):
            t = _NB * o + u

            @pl.when((o < _T // _NB - 1) | (u == 0))
            def _():
                gcopy(t + _NB - 1, (u + _NB - 1) % _NB).start()

            gcopy(t, u).wait()

            @pl.when(o >= 1)
            def _():
                ocopy(t - _NB, u).wait()

            transpose(u)
            ocopy(t, u).start()
        return carry

    lax.fori_loop(0, _T // _NB, body, 0)
    for t in range(_T - _NB, _T):
        ocopy(t, t % _NB).wait()


def kernel(captions, lengths, emb_weight):
    cap_t = captions.T
    out5d = _lookup_kernel(cap_t, emb_weight)
    x = out5d.transpose(2, 4, 0, 1, 3).reshape(_B, _T, _D)
    return x, lengths


# speedup vs baseline: 1.1935x; 1.0019x over previous
import functools

import jax
import jax.numpy as jnp
from jax import lax
from jax.experimental import pallas as pl
from jax.experimental.pallas import tpu as pltpu
from jax.experimental.pallas import tpu_sc as plsc

_B = 4096
_T = 50
_D = 64
_NW = 32
_BT = _B // 128
_NR = 10
_NO = 5
_LK = 8

_mesh = plsc.VectorSubcoreMesh(core_axis_name="c", subcore_axis_name="s")


@functools.partial(
    pl.kernel,
    mesh=_mesh,
    out_type=jax.ShapeDtypeStruct((_T, _D // 8, _BT, 8, 128), jnp.float32),
    scratch_types=[
        pltpu.VMEM((_T, 128), jnp.int32),
        [pltpu.VMEM((128, _D), jnp.float32) for _ in range(_NR)],
        [pltpu.VMEM((_D // 8, 8, 128), jnp.float32) for _ in range(_NO)],
        pltpu.SemaphoreType.DMA,
        pltpu.SemaphoreType.DMA((_NR,)),
        pltpu.SemaphoreType.DMA((_NO,)),
    ],
    compiler_params=pltpu.CompilerParams(
        use_tc_tiling_on_sc=False,
        needs_layout_passes=False,
        disable_bounds_checks=True,
    ),
)
def _lookup_kernel(cap_hbm, tbl_hbm, out_hbm, idx_v, rows_v, obuf_v,
                   isem, gsem, osem):
    w = lax.axis_index("s") * 2 + lax.axis_index("c")
    riota = lax.iota(jnp.int32, 16)
    rks = [riota + 16 * g for g in range(8)]

    pltpu.make_async_copy(
        cap_hbm.at[:, pl.ds(w * 128, 128)], idx_v, isem
    ).start()
    pltpu.make_async_copy(
        cap_hbm.at[:, pl.ds(w * 128, 128)], idx_v, isem
    ).wait()

    def gcopy(t, u):
        return pltpu.make_async_copy(
            tbl_hbm.at[idx_v.at[t]], rows_v[u], gsem.at[u]
        )

    def ocopy(t, u):
        return pltpu.make_async_copy(
            obuf_v[u], out_hbm.at[t, :, w], osem.at[u]
        )

    def transpose(ru, ou):
        def dbody(i, carry):
            vals = []
            for k in range(2):
                d = 2 * i + k
                dvec = jnp.full((16,), 0, jnp.int32) + d
                vals.append(
                    [plsc.load_gather(rows_v[ru], [rks[g], dvec])
                     for g in range(8)]
                )
            for k in range(2):
                d = 2 * i + k
                dt = lax.shift_right_logical(d, 3)
                ds_ = d & 7
                for g in range(8):
                    obuf_v[ou][dt, ds_, pl.ds(16 * g, 16)] = vals[k][g]
            return carry

        lax.fori_loop(0, _D // 2, dbody, 0)

    for t in range(_LK):
        gcopy(t, t).start()

    def body(o, carry):
        for u in range(_NR):
            t = _NR * o + u

            if u < _NR - _LK:
                gcopy(t + _LK, (u + _LK) % _NR).start()
            else:
                @pl.when(o < _T // _NR - 1)
                def _():
                    gcopy(t + _LK, (u + _LK) % _NR).start()

            gcopy(t, u).wait()

            if u >= _NO:
                ocopy(t - _NO, (u - _NO) % _NO).wait()
            else:
                @pl.when(o >= 1)
                def _():
                    ocopy(t - _NO, u % _NO).wait()

            transpose(u, u % _NO)
            ocopy(t, u % _NO).start()
        return carry

    lax.fori_loop(0, _T // _NR, body, 0)
    for t in range(_T - _NO, _T):
        ocopy(t, t % _NO).wait()


def kernel(captions, lengths, emb_weight):
    cap_t = captions.T
    out5d = _lookup_kernel(cap_t, emb_weight)
    x = out5d.transpose(2, 4, 0, 1, 3).reshape(_B, _T, _D)
    return x, lengths


# speedup vs baseline: 1.2950x; 1.0850x over previous
import functools

import jax
import jax.numpy as jnp
from jax import lax
from jax.experimental import pallas as pl
from jax.experimental.pallas import tpu as pltpu
from jax.experimental.pallas import tpu_sc as plsc

_B = 4096
_T = 50
_D = 64
_N = _B * _T
_NW = 32
_PER_W = _N // _NW
_CHUNK = 128
_NCH = _PER_W // _CHUNK
_NB = 4
_LK = 3

_mesh = plsc.VectorSubcoreMesh(core_axis_name="c", subcore_axis_name="s")


@functools.partial(
    pl.kernel,
    mesh=_mesh,
    out_type=jax.ShapeDtypeStruct((_N, _D), jnp.float32),
    scratch_types=[
        pltpu.VMEM((_PER_W,), jnp.int32),
        [pltpu.VMEM((_CHUNK, _D), jnp.float32) for _ in range(_NB)],
        pltpu.SemaphoreType.DMA((_NB,)),
        pltpu.SemaphoreType.DMA((_NB,)),
    ],
    compiler_params=pltpu.CompilerParams(
        use_tc_tiling_on_sc=False,
        disable_bounds_checks=True,
    ),
)
def _gather_kernel(idx_hbm, table_hbm, out_hbm, idx_v, rows_v, sem, osem):
    wid = lax.axis_index("s") * 2 + lax.axis_index("c")
    base = wid * _PER_W
    pltpu.sync_copy(idx_hbm.at[pl.ds(base, _PER_W)], idx_v)

    def gather(j):
        return pltpu.make_async_copy(
            table_hbm.at[idx_v.at[pl.ds(j * _CHUNK, _CHUNK)]],
            rows_v[j % _NB],
            sem.at[j % _NB],
        )

    def put(j):
        return pltpu.make_async_copy(
            rows_v[j % _NB],
            out_hbm.at[pl.ds(base + j * _CHUNK, _CHUNK)],
            osem.at[j % _NB],
        )

    for j in range(_LK):
        gather(j).start()
    for j in range(_NCH):
        gather(j).wait()
        put(j).start()
        jn = j + _LK
        if jn < _NCH:
            if jn >= _NB:
                put(jn - _NB).wait()
            gather(jn).start()
    for j in range(_NCH - _NB, _NCH):
        put(j).wait()


def kernel(captions, lengths, emb_weight):
    idx = captions.reshape(_N)
    out = _gather_kernel(idx, emb_weight)
    return out.reshape(_B, _T, _D), lengths


# speedup vs baseline: 1.2953x; 1.0003x over previous
import functools

import jax
import jax.numpy as jnp
from jax import lax
from jax.experimental import pallas as pl
from jax.experimental.pallas import tpu as pltpu
from jax.experimental.pallas import tpu_sc as plsc

_B = 4096
_T = 50
_D = 64
_N = _B * _T
_NW = 32
_PER_W = _N // _NW
_CHUNK = 128
_NCH = _PER_W // _CHUNK
_NB = 6
_LK = 5

_mesh = plsc.VectorSubcoreMesh(core_axis_name="c", subcore_axis_name="s")


@functools.partial(
    pl.kernel,
    mesh=_mesh,
    out_type=jax.ShapeDtypeStruct((_N, _D), jnp.float32),
    scratch_types=[
        pltpu.VMEM((_PER_W,), jnp.int32),
        [pltpu.VMEM((_CHUNK, _D), jnp.float32) for _ in range(_NB)],
        pltpu.SemaphoreType.DMA((_NB,)),
        pltpu.SemaphoreType.DMA((_NB,)),
    ],
    compiler_params=pltpu.CompilerParams(
        use_tc_tiling_on_sc=False,
        disable_bounds_checks=True,
    ),
)
def _gather_kernel(idx_hbm, table_hbm, out_hbm, idx_v, rows_v, sem, osem):
    wid = lax.axis_index("s") * 2 + lax.axis_index("c")
    base = wid * _PER_W
    pltpu.sync_copy(idx_hbm.at[pl.ds(base, _PER_W)], idx_v)

    def gather(j):
        return pltpu.make_async_copy(
            table_hbm.at[idx_v.at[pl.ds(j * _CHUNK, _CHUNK)]],
            rows_v[j % _NB],
            sem.at[j % _NB],
        )

    def put(j):
        return pltpu.make_async_copy(
            rows_v[j % _NB],
            out_hbm.at[pl.ds(base + j * _CHUNK, _CHUNK)],
            osem.at[j % _NB],
        )

    for j in range(_LK):
        gather(j).start()
    for j in range(_NCH):
        gather(j).wait()
        put(j).start()
        jn = j + _LK
        if jn < _NCH:
            if jn >= _NB:
                put(jn - _NB).wait()
            gather(jn).start()
    for j in range(_NCH - _NB, _NCH):
        put(j).wait()


def kernel(captions, lengths, emb_weight):
    idx = captions.reshape(_N)
    out = _gather_kernel(idx, emb_weight)
    return out.reshape(_B, _T, _D), lengths
